# Initial kernel scaffold; baseline (speedup 1.0000x reference)
import jax
import jax.numpy as jnp
from jax import lax
from jax.experimental import pallas as pl
from jax.experimental.pallas import tpu as pltpu
from jax.experimental.pallas import tpu_sc as plsc

N = 10000
E = 320000
D_IN = 128
D_H = 256
D_OUT = 64

T = 16
EPT = E // T
CHUNK = 2000
NCHUNK = EPT // CHUNK
VPC = CHUNK // 16
SCAP = N + 16
NVEC = N // 16
MCAP = CHUNK + 16
NPAD = 10240

_mesh = plsc.VectorSubcoreMesh(
    core_axis_name="c", subcore_axis_name="s", num_cores=1, num_subcores=T
)


def _rsqrt(x):
  i = plsc.bitcast(x, jnp.int32)
  y = plsc.bitcast(jnp.int32(0x5F3759DF) - (i >> 1), jnp.float32)
  for _ in range(4):
    y = y * (1.5 - 0.5 * x * y * y)
  return y


def _sc_body(
    ei_hbm, x_hbm, w1_hbm, b1_hbm,
    z_hbm, l0_hbm,
    dbuf, sbuf, dis_v, flag_v, l0buf, slotnode_v, w_v,
    msrc, mslot, mnrm, idxg, slotg, nrmg, rows_v,
    w1_v, b1_v, arow, zacc, zrow, vec16, cntall_v, degbuf, zp_v,
    deg_sh, dis_sh, flag_sh, slotnode_sh, w_sh, meta_sh, cnt_sh,
    agg_sh, zpart_sh,
):
  t = lax.axis_index("s")
  iota = lax.iota(jnp.int32, 16)
  fzero16 = jnp.zeros((16,), jnp.float32)
  izero16 = jnp.zeros((16,), jnp.int32)

  def _z(i, c):
    dis_v[pl.ds(i * 16, 16)] = fzero16
    flag_v[pl.ds(i * 16, 16)] = izero16
    return c
  lax.fori_loop(0, NVEC, _z, 0)

  def _z2(i, c):
    w_v[pl.ds(i * 16, 16)] = fzero16
    slotnode_v[pl.ds(i * 16, 16)] = izero16
    return c
  lax.fori_loop(0, SCAP // 16, _z2, 0)

  def _z3(i, c):
    zrow[pl.ds(i * 16, 16)] = fzero16
    return c
  lax.fori_loop(0, 8, _z3, 0)

  def _z4(i, c):
    zacc[pl.ds(i * 16, 16)] = fzero16
    return c
  lax.fori_loop(0, 16, _z4, 0)

  plsc.subcore_barrier()

  @pl.when(t == 0)
  def _init_deg():
    pltpu.sync_copy(dis_v, deg_sh.at[pl.ds(0, N)])
  plsc.subcore_barrier()

  def _chunk_a(c, cnt0):
    base = pl.multiple_of(t * EPT + c * CHUNK, 8)
    pltpu.sync_copy(ei_hbm.at[1, pl.ds(base, CHUNK)], dbuf)
    pltpu.sync_copy(ei_hbm.at[0, pl.ds(base, CHUNK)], sbuf)

    def _vec(i, cnt0):
      d = dbuf[pl.ds(i * 16, 16)]
      cntv, lastm = plsc.scan_count(d)
      plsc.addupdate_scatter(
          dis_v, [d], cntv.astype(jnp.float32), mask=lastm)
      m = d == 0
      npos = jnp.sum(m.astype(jnp.int32))

      def _found(cc):
        s = sbuf[pl.ds(i * 16, 16)]
        pos = plsc.cumsum(m.astype(jnp.int32)) - 1 + cc
        plsc.store_scatter(l0buf, [pos], s, mask=m)
        return cc + npos

      return lax.cond(npos > 0, _found, lambda cc: cc, cnt0)

    return lax.fori_loop(0, VPC, _vec, cnt0)

  cnt0 = lax.fori_loop(0, NCHUNK, _chunk_a, jnp.int32(0))

  pltpu.sync_copy(dis_v, deg_sh.at[pl.ds(0, N)], add=True)
  pltpu.sync_copy(l0buf, l0_hbm.at[pl.ds(pl.multiple_of(t * EPT, 8), EPT)])
  vec16[...] = jnp.full((16,), cnt0, jnp.int32)
  pltpu.sync_copy(vec16, cnt_sh.at[t])
  plsc.subcore_barrier()

  boff = pl.multiple_of(t * 640, 8)
  pltpu.sync_copy(deg_sh.at[pl.ds(boff, 640)], degbuf)

  def _dis(i, c):
    dv = degbuf[pl.ds(i * 16, 16)] + 1.0
    degbuf[pl.ds(i * 16, 16)] = _rsqrt(dv)
    return c
  lax.fori_loop(0, 40, _dis, 0)
  pltpu.sync_copy(degbuf, dis_sh.at[pl.ds(boff, 640)])
  plsc.subcore_barrier()
  pltpu.sync_copy(dis_sh.at[pl.ds(0, N)], dis_v)

  @pl.when(t == 0)
  def _dedup():
    pltpu.sync_copy(cnt_sh, cntall_v)
    flag_v[0] = jnp.int32(1)

    def _tile(tt, ns):
      cnt_t = cntall_v[tt, 0]

      def _chunk(c, ns):
        cbase = pl.multiple_of(tt * EPT + c * CHUNK, 8)
        pltpu.sync_copy(l0_hbm.at[pl.ds(cbase, CHUNK)], dbuf)
        kmax = jnp.minimum(jnp.int32(CHUNK), cnt_t - c * CHUNK)

        def _k(k, ns):
          s = dbuf[k]
          f = flag_v[s]
          isnew = (f == 0).astype(jnp.int32)
          slot = jnp.where(f == 0, ns, f - 1)
          flag_v[s] = slot + 1
          slotnode_v[slot] = s
          w_v[slot] = w_v[slot] + dis_v[s]
          return ns + isnew

        return lax.fori_loop(0, kmax, _k, ns)

      nchunks = (cnt_t + CHUNK - 1) // CHUNK
      return lax.fori_loop(0, nchunks, _chunk, ns)

    ns = lax.fori_loop(0, T, _tile, jnp.int32(1))
    pltpu.sync_copy(flag_v, flag_sh)
    pltpu.sync_copy(slotnode_v, slotnode_sh)
    pltpu.sync_copy(w_v, w_sh)
    vec16[...] = jnp.full((16,), ns, jnp.int32)
    pltpu.sync_copy(vec16, meta_sh)

  plsc.subcore_barrier()

  pltpu.sync_copy(flag_sh, flag_v)
  pltpu.sync_copy(slotnode_sh, slotnode_v)
  pltpu.sync_copy(w_sh, w_v)
  pltpu.sync_copy(meta_sh, vec16)
  nslots = vec16[0]

  nz = (nslots - t + T - 1) // T

  def _za(k, c):
    j = t + k * T
    pltpu.sync_copy(zrow, agg_sh.at[j])
    return c
  lax.fori_loop(0, nz, _za, 0)
  plsc.subcore_barrier()

  def _process16(srcv, slotv, nrmv):
    idxg[...] = srcv
    slotg[...] = slotv
    nrmg[...] = nrmv
    pltpu.sync_copy(x_hbm.at[idxg], rows_v)

    def _row(l, c):
      nl = nrmg[l]

      def _b(b, c2):
        v = rows_v[l, pl.ds(b * 16, 16)]
        rows_v[l, pl.ds(b * 16, 16)] = v * nl
        return c2
      lax.fori_loop(0, 8, _b, 0)
      return c
    lax.fori_loop(0, 16, _row, 0)
    pltpu.sync_copy(rows_v, agg_sh.at[slotg], add=True)

  def _chunk_e(c, cc):
    base = pl.multiple_of(t * EPT + c * CHUNK, 8)
    pltpu.sync_copy(ei_hbm.at[1, pl.ds(base, CHUNK)], dbuf)
    pltpu.sync_copy(ei_hbm.at[0, pl.ds(base, CHUNK)], sbuf)

    def _vec(i, mcnt):
      d = dbuf[pl.ds(i * 16, 16)]
      f = plsc.load_gather(flag_v, [d])
      m = f > 0
      npos = jnp.sum(m.astype(jnp.int32))

      def _found(mc):
        s = sbuf[pl.ds(i * 16, 16)]
        nrm = plsc.load_gather(dis_v, [s]) * plsc.load_gather(dis_v, [d])
        pos = plsc.cumsum(m.astype(jnp.int32)) - 1 + mc
        plsc.store_scatter(msrc, [pos], s, mask=m)
        plsc.store_scatter(mslot, [pos], f - 1, mask=m)
        plsc.store_scatter(mnrm, [pos], nrm, mask=m)
        return mc + npos

      return lax.cond(npos > 0, _found, lambda mc: mc, mcnt)

    mcnt = lax.fori_loop(0, VPC, _vec, jnp.int32(0))

    flo = (mcnt // 16) * 16
    padm = (iota + flo) >= mcnt
    plsc.store_scatter(msrc, [iota + flo], izero16, mask=padm)
    plsc.store_scatter(mslot, [iota + flo], izero16, mask=padm)
    plsc.store_scatter(mnrm, [iota + flo], fzero16, mask=padm)

    nbat = (mcnt + 15) // 16

    def _bat(r, c2):
      _process16(
          msrc[pl.ds(r * 16, 16)],
          mslot[pl.ds(r * 16, 16)],
          mnrm[pl.ds(r * 16, 16)],
      )
      return c2
    lax.fori_loop(0, nbat, _bat, 0)
    return cc

  lax.fori_loop(0, NCHUNK, _chunk_e, 0)

  nv_slots = (nslots + 15) // 16
  nk = (nv_slots - t + T - 1) // T

  def _selfk(k, c):
    v = t + k * T
    jvec = iota + v * 16
    m = jvec < nslots
    nodes = plsc.load_gather(slotnode_v, [jvec], mask=m)
    nodes = jnp.where(m, nodes, 0)
    dv = plsc.load_gather(dis_v, [nodes])
    nrm = jnp.where(m, dv * dv, fzero16)
    slots = jnp.where(m, jvec, 0)
    _process16(nodes, slots, nrm)
    return c
  lax.fori_loop(0, nk, _selfk, 0)
  plsc.subcore_barrier()

  pltpu.sync_copy(w1_hbm, w1_v)
  pltpu.sync_copy(b1_hbm, b1_v)
  dis0 = dis_v[0]

  def _slot(k, c):
    j = t + k * T
    pltpu.sync_copy(agg_sh.at[j], arow)
    accs = tuple(b1_v[pl.ds(jb * 16, 16)] for jb in range(16))

    def _kk(kk, accs):
      ak = arow[kk]
      return tuple(
          accs[jb] + ak * w1_v[kk, pl.ds(jb * 16, 16)] for jb in range(16)
      )
    accs = lax.fori_loop(0, D_IN, _kk, accs)

    wt = dis0 * w_v[j] + jnp.where(j == 0, dis0 * dis0, jnp.float32(0.0))
    for jb in range(16):
      h = jnp.maximum(accs[jb], 0.0)
      zacc[pl.ds(jb * 16, 16)] = zacc[pl.ds(jb * 16, 16)] + wt * h
    return c

  lax.fori_loop(0, nz, _slot, 0)
  pltpu.sync_copy(zacc, zpart_sh.at[t])
  plsc.subcore_barrier()

  @pl.when(t == 0)
  def _finish():
    pltpu.sync_copy(zpart_sh, zp_v)

    def _jb(jb, c):
      def _tt(tt, acc):
        return acc + zp_v[tt, pl.ds(jb * 16, 16)]
      acc = lax.fori_loop(0, T, _tt, fzero16)
      zacc[pl.ds(jb * 16, 16)] = acc
      return c
    lax.fori_loop(0, 16, _jb, 0)
    pltpu.sync_copy(zacc, z_hbm)


_sc_kernel = pl.kernel(
    _sc_body,
    out_type=(
        jax.ShapeDtypeStruct((D_H,), jnp.float32),
        jax.ShapeDtypeStruct((E,), jnp.int32),
    ),
    mesh=_mesh,
    compiler_params=pltpu.CompilerParams(needs_layout_passes=False),
    scratch_types=[
        pltpu.VMEM((CHUNK,), jnp.int32),
        pltpu.VMEM((CHUNK,), jnp.int32),
        pltpu.VMEM((N,), jnp.float32),
        pltpu.VMEM((N,), jnp.int32),
        pltpu.VMEM((EPT,), jnp.int32),
        pltpu.VMEM((SCAP,), jnp.int32),
        pltpu.VMEM((SCAP,), jnp.float32),
        pltpu.VMEM((MCAP,), jnp.int32),
        pltpu.VMEM((MCAP,), jnp.int32),
        pltpu.VMEM((MCAP,), jnp.float32),
        pltpu.VMEM((16,), jnp.int32),
        pltpu.VMEM((16,), jnp.int32),
        pltpu.VMEM((16,), jnp.float32),
        pltpu.VMEM((16, D_IN), jnp.float32),
        pltpu.VMEM((D_IN, D_H), jnp.float32),
        pltpu.VMEM((D_H,), jnp.float32),
        pltpu.VMEM((D_IN,), jnp.float32),
        pltpu.VMEM((D_H,), jnp.float32),
        pltpu.VMEM((D_IN,), jnp.float32),
        pltpu.VMEM((16,), jnp.int32),
        pltpu.VMEM((T, 16), jnp.int32),
        pltpu.VMEM((640,), jnp.float32),
        pltpu.VMEM((T, D_H), jnp.float32),
        pltpu.VMEM_SHARED((NPAD,), jnp.float32),
        pltpu.VMEM_SHARED((NPAD,), jnp.float32),
        pltpu.VMEM_SHARED((N,), jnp.int32),
        pltpu.VMEM_SHARED((SCAP,), jnp.int32),
        pltpu.VMEM_SHARED((SCAP,), jnp.float32),
        pltpu.VMEM_SHARED((16,), jnp.int32),
        pltpu.VMEM_SHARED((T, 16), jnp.int32),
        pltpu.VMEM_SHARED((SCAP, D_IN), jnp.float32),
        pltpu.VMEM_SHARED((T, D_H), jnp.float32),
    ],
)


def _head_body(z_ref, w2_ref, b2_ref, wh1_ref, bh1_ref, wh2_ref, bh2_ref,
               o_ref):
  z = z_ref[...]
  h2 = jnp.maximum(
      jnp.dot(z, w2_ref[...], preferred_element_type=jnp.float32)
      + b2_ref[...], 0.0)
  hid = jnp.maximum(
      jnp.dot(h2, wh1_ref[...], preferred_element_type=jnp.float32)
      + bh1_ref[...], 0.0)
  o_ref[...] = (
      jnp.dot(hid, wh2_ref[...], preferred_element_type=jnp.float32)
      + bh2_ref[...])


_head_call = pl.pallas_call(
    _head_body,
    out_shape=jax.ShapeDtypeStruct((1, D_OUT), jnp.float32),
)


def kernel(x, edge_index, W1, b1, W2, b2, Wh1, bh1, Wh2, bh2):
  z, _ = _sc_kernel(edge_index, x, W1, b1)
  q = _head_call(
      z.reshape(1, D_H), W2, b2.reshape(1, D_H),
      Wh1, bh1.reshape(1, D_H), Wh2, bh2.reshape(1, D_OUT))
  return q.reshape(D_OUT)


# speedup vs baseline: 60.2466x; 60.2466x over previous
import jax
import jax.numpy as jnp
from jax import lax
from jax.experimental import pallas as pl
from jax.experimental.pallas import tpu as pltpu
from jax.experimental.pallas import tpu_sc as plsc

N = 10000
E = 320000
D_IN = 128
D_H = 256
D_OUT = 64

T = 16
EPT = E // T
CHUNK = 2000
NCHUNK = EPT // CHUNK
VPC = CHUNK // 16
SCAP = N + 16
NVEC = N // 16
MCAP = CHUNK + 16
NPAD = 10240
SMAX = 1024

_mesh = plsc.VectorSubcoreMesh(
    core_axis_name="c", subcore_axis_name="s", num_cores=1, num_subcores=T
)


def _rsqrt(x):
  i = plsc.bitcast(x, jnp.int32)
  y = plsc.bitcast(jnp.int32(0x5F3759DF) - (i >> 1), jnp.float32)
  for _ in range(4):
    y = y * (1.5 - 0.5 * x * y * y)
  return y


def _sc_body(
    src_hbm, dst_hbm, x_hbm, w1_hbm, b1_hbm,
    z_hbm, l0_hbm, hist_hbm,
    dbuf, sbuf, dis_v, flag_v, l0buf, slotnode_v, w_v,
    msrc, mslot, mnrm, idxg, slotg, rows_v,
    w1_v, b1_v, zblk, zfull, vec16, cntall_v, degbuf, hbuf,
    dis_sh, flag_sh, slotnode_sh, w_sh, meta_sh, cnt_sh, agg_sh, z_sh,
):
  t = lax.axis_index("s")
  iota = lax.iota(jnp.int32, 16)
  fzero16 = jnp.zeros((16,), jnp.float32)
  izero16 = jnp.zeros((16,), jnp.int32)

  def _z(i, c):
    dis_v[pl.ds(i * 16, 16)] = fzero16
    flag_v[pl.ds(i * 16, 16)] = izero16
    return c
  lax.fori_loop(0, NVEC, _z, 0)

  def _z2(i, c):
    w_v[pl.ds(i * 16, 16)] = fzero16
    slotnode_v[pl.ds(i * 16, 16)] = izero16
    return c
  lax.fori_loop(0, SCAP // 16, _z2, 0)

  for l in range(16):
    def _zr(b, c, l=l):
      rows_v[l, pl.ds(b * 16, 16)] = fzero16
      return c
    lax.fori_loop(0, 8, _zr, 0)
  zblk[...] = fzero16

  def _chunk_a(c, cnt0):
    base = pl.multiple_of((t * NCHUNK + c) * CHUNK, 8)
    pltpu.sync_copy(dst_hbm.at[pl.ds(base, CHUNK)], dbuf.at[pl.ds(0, CHUNK)])
    pltpu.sync_copy(src_hbm.at[pl.ds(base, CHUNK)], sbuf)

    def _vec(i, cnt0):
      d = dbuf[pl.ds(i * 16, 16)]
      cntv, lastm = plsc.scan_count(d)
      plsc.addupdate_scatter(
          dis_v, [d], cntv.astype(jnp.float32), mask=lastm)
      m = d == 0
      npos = jnp.sum(m.astype(jnp.int32))

      def _found(cc):
        s = sbuf[pl.ds(i * 16, 16)]
        pos = plsc.cumsum(m.astype(jnp.int32)) - 1 + cc
        plsc.store_scatter(l0buf, [pos], s, mask=m)
        return cc + npos

      return lax.cond(npos > 0, _found, lambda cc: cc, cnt0)

    return lax.fori_loop(0, VPC, _vec, cnt0)

  cnt0 = lax.fori_loop(0, NCHUNK, _chunk_a, jnp.int32(0))

  pltpu.sync_copy(dis_v.at[pl.ds(0, N)],
                  hist_hbm.at[pl.ds(pl.multiple_of(t * NPAD, 8), N)])
  pltpu.sync_copy(l0buf, l0_hbm.at[pl.ds(pl.multiple_of(t * EPT, 8), EPT)])
  vec16[...] = jnp.full((16,), cnt0, jnp.int32)
  pltpu.sync_copy(vec16, cnt_sh.at[pl.ds(pl.multiple_of(t * 16, 8), 16)])
  plsc.subcore_barrier()

  def _zdeg(i, c):
    degbuf[pl.ds(i * 16, 16)] = fzero16
    return c
  lax.fori_loop(0, 40, _zdeg, 0)

  def _red(tt, c):
    hoff = pl.multiple_of(tt * NPAD + t * 640, 8)
    pltpu.sync_copy(hist_hbm.at[pl.ds(hoff, 640)], hbuf)

    def _acc(i, c2):
      degbuf[pl.ds(i * 16, 16)] = (
          degbuf[pl.ds(i * 16, 16)] + hbuf[pl.ds(i * 16, 16)])
      return c2
    lax.fori_loop(0, 40, _acc, 0)
    return c
  lax.fori_loop(0, T, _red, 0)

  def _dis(i, c):
    dv = degbuf[pl.ds(i * 16, 16)] + 1.0
    degbuf[pl.ds(i * 16, 16)] = _rsqrt(dv)
    return c
  lax.fori_loop(0, 40, _dis, 0)
  pltpu.sync_copy(degbuf, dis_sh.at[pl.ds(pl.multiple_of(t * 640, 8), 640)])
  plsc.subcore_barrier()
  pltpu.sync_copy(dis_sh.at[pl.ds(0, N)], dis_v.at[pl.ds(0, N)])

  lane0 = iota == 0

  def _sstore(ref, idx, val):
    plsc.store_scatter(
        ref, [jnp.full((16,), idx, jnp.int32)],
        jnp.full((16,), val, ref.dtype), mask=lane0)

  @pl.when(t == 0)
  def _dedup():
    pltpu.sync_copy(cnt_sh, cntall_v)
    _sstore(flag_v, jnp.int32(0), jnp.int32(1))

    def _tile(tt, ns):
      cnt_t = cntall_v[pl.ds(tt * 16, 16)][0]

      def _chunk(c, ns):
        cbase = pl.multiple_of((tt * NCHUNK + c) * CHUNK, 8)
        pltpu.sync_copy(l0_hbm.at[pl.ds(cbase, CHUNK)],
                        dbuf.at[pl.ds(0, CHUNK)])
        kmax = jnp.minimum(jnp.int32(CHUNK), cnt_t - c * CHUNK)

        def _k(k, ns):
          s = dbuf[pl.ds(k, 16)][0]
          f = flag_v[pl.ds(s, 16)][0]
          isnew = (f == 0).astype(jnp.int32)
          slot = jnp.where(f == 0, ns, f - 1)
          _sstore(flag_v, s, slot + 1)
          _sstore(slotnode_v, slot, s)
          wnew = w_v[pl.ds(slot, 16)][0] + dis_v[pl.ds(s, 16)][0]
          _sstore(w_v, slot, wnew)
          return ns + isnew

        return lax.fori_loop(0, kmax, _k, ns)

      nchunks = (cnt_t + CHUNK - 1) // CHUNK
      return lax.fori_loop(0, nchunks, _chunk, ns)

    ns = lax.fori_loop(0, T, _tile, jnp.int32(1))
    pltpu.sync_copy(flag_v.at[pl.ds(0, N)], flag_sh)
    pltpu.sync_copy(slotnode_v, slotnode_sh)
    pltpu.sync_copy(w_v, w_sh)
    vec16[...] = jnp.full((16,), ns, jnp.int32)
    pltpu.sync_copy(vec16, meta_sh)

  plsc.subcore_barrier()

  pltpu.sync_copy(flag_sh, flag_v.at[pl.ds(0, N)])
  pltpu.sync_copy(slotnode_sh, slotnode_v)
  pltpu.sync_copy(w_sh, w_v)
  pltpu.sync_copy(meta_sh, vec16)
  nslots = vec16[...][0]
  dis0 = dis_v[pl.ds(0, 16)][0]
  pltpu.sync_copy(w1_hbm.at[pl.ds(pl.multiple_of(t * (D_IN * 16), 8),
                                  D_IN * 16)], w1_v)
  pltpu.sync_copy(b1_hbm.at[pl.ds(pl.multiple_of(t * 16, 8), 16)], b1_v)

  def _process16(srcv, slotv, nrmv):
    idxg[...] = srcv
    slotg[...] = slotv
    pltpu.sync_copy(x_hbm.at[idxg], rows_v)

    for l in range(16):
      nl = nrmv[l]

      def _b(b, c2, l=l, nl=nl):
        v = rows_v[l, pl.ds(b * 16, 16)]
        rows_v[l, pl.ds(b * 16, 16)] = v * nl
        return c2
      lax.fori_loop(0, 8, _b, 0)
    pltpu.sync_copy(rows_v, agg_sh.at[slotg], add=True)

  ngroups = (nslots + SMAX - 1) // SMAX

  def _group(g, c):
    glo = g * SMAX
    gcount = jnp.minimum(nslots - glo, jnp.int32(SMAX))

    for l in range(16):
      def _zr2(b, c2, l=l):
        rows_v[l, pl.ds(b * 16, 16)] = fzero16
        return c2
      lax.fori_loop(0, 8, _zr2, 0)

    mv = (gcount + 15) // 16

    def _za(k, c2):
      mchunk = k * 16 + t
      rvec = mchunk * 16 + iota
      rz = jnp.where(rvec < gcount, rvec, jnp.int32(SMAX))
      slotg[...] = rz
      pltpu.sync_copy(rows_v, agg_sh.at[slotg])
      return c2
    lax.fori_loop(0, jnp.maximum(0, (mv - t + 15) // 16), _za, 0)
    plsc.subcore_barrier()

    def _chunk_e(cch, cc):
      base = pl.multiple_of((t * NCHUNK + cch) * CHUNK, 8)
      pltpu.sync_copy(dst_hbm.at[pl.ds(base, CHUNK)],
                      dbuf.at[pl.ds(0, CHUNK)])
      pltpu.sync_copy(src_hbm.at[pl.ds(base, CHUNK)], sbuf)

      def _vec(i, mcnt):
        d = dbuf[pl.ds(i * 16, 16)]
        f = plsc.load_gather(flag_v, [d])
        gs = f - 1 - glo
        m = (f > 0) & (gs >= 0) & (gs < gcount)
        npos = jnp.sum(m.astype(jnp.int32))

        def _found(mc):
          s = sbuf[pl.ds(i * 16, 16)]
          nrm = plsc.load_gather(dis_v, [s]) * plsc.load_gather(dis_v, [d])
          pos = plsc.cumsum(m.astype(jnp.int32)) - 1 + mc
          plsc.store_scatter(msrc, [pos], s, mask=m)
          plsc.store_scatter(mslot, [pos], gs, mask=m)
          plsc.store_scatter(mnrm, [pos], nrm, mask=m)
          return mc + npos

        return lax.cond(npos > 0, _found, lambda mc: mc, mcnt)

      mcnt = lax.fori_loop(0, VPC, _vec, jnp.int32(0))

      flo = (mcnt // 16) * 16
      padm = (iota + flo) >= mcnt
      plsc.store_scatter(msrc, [iota + flo], izero16, mask=padm)
      plsc.store_scatter(mslot, [iota + flo],
                         jnp.full((16,), SMAX, jnp.int32), mask=padm)
      plsc.store_scatter(mnrm, [iota + flo], fzero16, mask=padm)

      def _bat(r, c2):
        _process16(
            msrc[pl.ds(r * 16, 16)],
            mslot[pl.ds(r * 16, 16)],
            mnrm[pl.ds(r * 16, 16)],
        )
        return c2
      lax.fori_loop(0, (mcnt + 15) // 16, _bat, 0)
      return cc

    lax.fori_loop(0, NCHUNK, _chunk_e, 0)

    gv = (gcount + 15) // 16

    def _selfk(k, c2):
      v = k * 16 + t
      gslot = v * 16 + iota
      jvec = glo + gslot
      m = gslot < gcount
      nodes = plsc.load_gather(slotnode_v, [jvec], mask=m)
      nodes = jnp.where(m, nodes, 0)
      dv = plsc.load_gather(dis_v, [nodes])
      nrm = jnp.where(m, dv * dv, fzero16)
      slots = jnp.where(m, gslot, jnp.int32(SMAX))
      _process16(nodes, slots, nrm)
      return c2
    lax.fori_loop(0, jnp.maximum(0, (gv - t + 15) // 16), _selfk, 0)
    plsc.subcore_barrier()

    def _fb(r0, c2):
      rvec = r0 * 16 + iota
      rz = jnp.where(rvec < gcount, rvec, 0)
      idxg[...] = rz
      pltpu.sync_copy(agg_sh.at[idxg], rows_v)
      zreg = zblk[...]
      for l in range(16):
        acc = b1_v[...]

        def _kv(kv, acc, l=l):
          av = rows_v[l, pl.ds(kv * 16, 16)]
          for lane in range(16):
            acc = acc + av[lane] * w1_v[pl.ds((kv * 16 + lane) * 16, 16)]
          return acc
        acc = lax.fori_loop(0, D_IN // 16, _kv, acc)
        h = jnp.maximum(acc, 0.0)
        j = glo + r0 * 16 + l
        valid = (r0 * 16 + l < gcount).astype(jnp.float32)
        wj = w_v[pl.ds(j, 16)][0]
        wt = (dis0 * wj
              + jnp.where(j == 0, dis0 * dis0, jnp.float32(0.0))) * valid
        zreg = zreg + wt * h
      zblk[...] = zreg
      return c2
    lax.fori_loop(0, (gcount + 15) // 16, _fb, 0)
    plsc.subcore_barrier()
    return c

  lax.fori_loop(0, ngroups, _group, 0)

  pltpu.sync_copy(zblk, z_sh.at[pl.ds(pl.multiple_of(t * 16, 8), 16)])
  plsc.subcore_barrier()

  @pl.when(t == 0)
  def _finish():
    pltpu.sync_copy(z_sh, zfull)
    pltpu.sync_copy(zfull, z_hbm)


_sc_kernel = pl.kernel(
    _sc_body,
    out_type=(
        jax.ShapeDtypeStruct((D_H,), jnp.float32),
        jax.ShapeDtypeStruct((E,), jnp.int32),
        jax.ShapeDtypeStruct((T * NPAD,), jnp.float32),
    ),
    mesh=_mesh,
    compiler_params=pltpu.CompilerParams(needs_layout_passes=False),
    scratch_types=[
        pltpu.VMEM((MCAP,), jnp.int32),
        pltpu.VMEM((CHUNK,), jnp.int32),
        pltpu.VMEM((N + 16,), jnp.float32),
        pltpu.VMEM((N + 16,), jnp.int32),
        pltpu.VMEM((EPT,), jnp.int32),
        pltpu.VMEM((SCAP,), jnp.int32),
        pltpu.VMEM((SCAP,), jnp.float32),
        pltpu.VMEM((MCAP,), jnp.int32),
        pltpu.VMEM((MCAP,), jnp.int32),
        pltpu.VMEM((MCAP,), jnp.float32),
        pltpu.VMEM((16,), jnp.int32),
        pltpu.VMEM((16,), jnp.int32),
        pltpu.VMEM((16, D_IN), jnp.float32),
        pltpu.VMEM((D_IN * 16,), jnp.float32),
        pltpu.VMEM((16,), jnp.float32),
        pltpu.VMEM((16,), jnp.float32),
        pltpu.VMEM((D_H,), jnp.float32),
        pltpu.VMEM((16,), jnp.int32),
        pltpu.VMEM((T * 16,), jnp.int32),
        pltpu.VMEM((640,), jnp.float32),
        pltpu.VMEM((640,), jnp.float32),
        pltpu.VMEM_SHARED((NPAD,), jnp.float32),
        pltpu.VMEM_SHARED((N,), jnp.int32),
        pltpu.VMEM_SHARED((SCAP,), jnp.int32),
        pltpu.VMEM_SHARED((SCAP,), jnp.float32),
        pltpu.VMEM_SHARED((16,), jnp.int32),
        pltpu.VMEM_SHARED((T * 16,), jnp.int32),
        pltpu.VMEM_SHARED((SMAX + 8, D_IN), jnp.float32),
        pltpu.VMEM_SHARED((D_H,), jnp.float32),
    ],
)


def _head_body(z_ref, w2_ref, b2_ref, wh1_ref, bh1_ref, wh2_ref, bh2_ref,
               o_ref):
  z = z_ref[...]
  h2 = jnp.maximum(
      jnp.dot(z, w2_ref[...], preferred_element_type=jnp.float32)
      + b2_ref[...], 0.0)
  hid = jnp.maximum(
      jnp.dot(h2, wh1_ref[...], preferred_element_type=jnp.float32)
      + bh1_ref[...], 0.0)
  o_ref[...] = (
      jnp.dot(hid, wh2_ref[...], preferred_element_type=jnp.float32)
      + bh2_ref[...])


_head_call = pl.pallas_call(
    _head_body,
    out_shape=jax.ShapeDtypeStruct((1, D_OUT), jnp.float32),
)


def kernel(x, edge_index, W1, b1, W2, b2, Wh1, bh1, Wh2, bh2):
  w1_blocks = W1.reshape(D_IN, 16, 16).transpose(1, 0, 2).reshape(-1)
  z, _, _ = _sc_kernel(edge_index[0], edge_index[1], x, w1_blocks, b1)
  q = _head_call(
      z.reshape(1, D_H), W2, b2.reshape(1, D_H),
      Wh1, bh1.reshape(1, D_H), Wh2, bh2.reshape(1, D_OUT))
  return q.reshape(D_OUT)


# speedup vs baseline: 62.7226x; 1.0411x over previous
import jax
import jax.numpy as jnp
from jax import lax
from jax.experimental import pallas as pl
from jax.experimental.pallas import tpu as pltpu
from jax.experimental.pallas import tpu_sc as plsc

N = 10000
E = 320000
D_IN = 128
D_H = 256
D_OUT = 64

T = 16
EPT = E // T
CHUNK = 2000
NCHUNK = EPT // CHUNK
VPC = CHUNK // 16
SCAP = N + 16
NVEC = N // 16
MCAP = CHUNK + 16
NPAD = 10240
SMAX = 1024

_mesh = plsc.VectorSubcoreMesh(
    core_axis_name="c", subcore_axis_name="s", num_cores=1, num_subcores=T
)


def _rsqrt(x):
  i = plsc.bitcast(x, jnp.int32)
  y = plsc.bitcast(jnp.int32(0x5F3759DF) - (i >> 1), jnp.float32)
  for _ in range(4):
    y = y * (1.5 - 0.5 * x * y * y)
  return y


def _sc_body(
    src_hbm, dst_hbm, x_hbm, w1_hbm, b1_hbm,
    z_hbm, l0_hbm, hist_hbm,
    dbuf, sbuf, dis_v, flag_v, l0buf, slotnode_v, w_v,
    msrc, mslot, mnrm, idxg, slotg, rows_v,
    w1_v, b1_v, zblk, zfull, vec16, cntall_v, degbuf, hbuf, hsem,
    dis_sh, flag_sh, slotnode_sh, w_sh, meta_sh, cnt_sh, agg_sh, z_sh,
):
  t = lax.axis_index("s")
  iota = lax.iota(jnp.int32, 16)
  fzero16 = jnp.zeros((16,), jnp.float32)
  izero16 = jnp.zeros((16,), jnp.int32)

  def _z(i, c):
    dis_v[pl.ds(i * 16, 16)] = fzero16
    flag_v[pl.ds(i * 16, 16)] = izero16
    return c
  lax.fori_loop(0, NVEC, _z, 0)

  def _z2(i, c):
    w_v[pl.ds(i * 16, 16)] = fzero16
    slotnode_v[pl.ds(i * 16, 16)] = izero16
    return c
  lax.fori_loop(0, SCAP // 16, _z2, 0)

  for l in range(16):
    def _zr(b, c, l=l):
      rows_v[l, pl.ds(b * 16, 16)] = fzero16
      return c
    lax.fori_loop(0, 8, _zr, 0)
  zblk[...] = fzero16

  def _chunk_a(c, cnt0):
    base = pl.multiple_of((t * NCHUNK + c) * CHUNK, 8)
    pltpu.sync_copy(dst_hbm.at[pl.ds(base, CHUNK)], dbuf.at[pl.ds(0, CHUNK)])
    pltpu.sync_copy(src_hbm.at[pl.ds(base, CHUNK)], sbuf)

    def _vec(i, cnt0):
      d = dbuf[pl.ds(i * 16, 16)]
      cntv, lastm = plsc.scan_count(d)
      plsc.addupdate_scatter(
          dis_v, [d], cntv.astype(jnp.float32), mask=lastm)
      m = d == 0

      def _found(cc):
        s = sbuf[pl.ds(i * 16, 16)]
        pc = plsc.cumsum(m.astype(jnp.int32))
        pos = pc - 1 + cc
        plsc.store_scatter(l0buf, [pos], s, mask=m)
        return cc + pc[15]

      return lax.cond(jnp.any(m), _found, lambda cc: cc, cnt0)

    return lax.fori_loop(0, VPC, _vec, cnt0, unroll=2)

  cnt0 = lax.fori_loop(0, NCHUNK, _chunk_a, jnp.int32(0))

  pltpu.sync_copy(dis_v.at[pl.ds(0, N)],
                  hist_hbm.at[pl.ds(pl.multiple_of(t * NPAD, 8), N)])
  pltpu.sync_copy(l0buf, l0_hbm.at[pl.ds(pl.multiple_of(t * EPT, 8), EPT)])
  vec16[...] = jnp.full((16,), cnt0, jnp.int32)
  pltpu.sync_copy(vec16, cnt_sh.at[pl.ds(pl.multiple_of(t * 16, 8), 16)])
  plsc.subcore_barrier()

  copies = [
      pltpu.make_async_copy(
          hist_hbm.at[pl.ds(pl.multiple_of(tt * NPAD + t * 640, 8), 640)],
          hbuf.at[pl.ds(tt * 640, 640)], hsem)
      for tt in range(T)
  ]
  for cp in copies:
    cp.start()
  for cp in copies:
    cp.wait()

  def _acc(i, c2):
    acc = hbuf[pl.ds(i * 16, 16)]
    for tt in range(1, T):
      acc = acc + hbuf[pl.ds(tt * 640 + i * 16, 16)]
    degbuf[pl.ds(i * 16, 16)] = acc
    return c2
  lax.fori_loop(0, 40, _acc, 0)

  def _dis(i, c):
    dv = degbuf[pl.ds(i * 16, 16)] + 1.0
    degbuf[pl.ds(i * 16, 16)] = _rsqrt(dv)
    return c
  lax.fori_loop(0, 40, _dis, 0)
  pltpu.sync_copy(degbuf, dis_sh.at[pl.ds(pl.multiple_of(t * 640, 8), 640)])
  plsc.subcore_barrier()
  pltpu.sync_copy(dis_sh.at[pl.ds(0, N)], dis_v.at[pl.ds(0, N)])

  lane0 = iota == 0

  def _sstore(ref, idx, val):
    plsc.store_scatter(
        ref, [jnp.full((16,), idx, jnp.int32)],
        jnp.full((16,), val, ref.dtype), mask=lane0)

  @pl.when(t == 0)
  def _dedup():
    pltpu.sync_copy(cnt_sh, cntall_v)
    _sstore(flag_v, jnp.int32(0), jnp.int32(1))

    def _tile(tt, ns):
      cnt_t = cntall_v[pl.ds(tt * 16, 16)][0]

      def _chunk(c, ns):
        cbase = pl.multiple_of((tt * NCHUNK + c) * CHUNK, 8)
        pltpu.sync_copy(l0_hbm.at[pl.ds(cbase, CHUNK)],
                        dbuf.at[pl.ds(0, CHUNK)])
        kmax = jnp.minimum(jnp.int32(CHUNK), cnt_t - c * CHUNK)

        def _k(k, ns):
          s = dbuf[pl.ds(k, 16)][0]
          f = flag_v[pl.ds(s, 16)][0]
          isnew = (f == 0).astype(jnp.int32)
          slot = jnp.where(f == 0, ns, f - 1)
          _sstore(flag_v, s, slot + 1)
          _sstore(slotnode_v, slot, s)
          wnew = w_v[pl.ds(slot, 16)][0] + dis_v[pl.ds(s, 16)][0]
          _sstore(w_v, slot, wnew)
          return ns + isnew

        return lax.fori_loop(0, kmax, _k, ns)

      nchunks = (cnt_t + CHUNK - 1) // CHUNK
      return lax.fori_loop(0, nchunks, _chunk, ns)

    ns = lax.fori_loop(0, T, _tile, jnp.int32(1))
    pltpu.sync_copy(flag_v.at[pl.ds(0, N)], flag_sh)
    pltpu.sync_copy(slotnode_v, slotnode_sh)
    pltpu.sync_copy(w_v, w_sh)
    vec16[...] = jnp.full((16,), ns, jnp.int32)
    pltpu.sync_copy(vec16, meta_sh)

  plsc.subcore_barrier()

  pltpu.sync_copy(flag_sh, flag_v.at[pl.ds(0, N)])
  pltpu.sync_copy(slotnode_sh, slotnode_v)
  pltpu.sync_copy(w_sh, w_v)
  pltpu.sync_copy(meta_sh, vec16)
  nslots = vec16[...][0]
  dis0 = dis_v[pl.ds(0, 16)][0]
  pltpu.sync_copy(w1_hbm.at[pl.ds(pl.multiple_of(t * (D_IN * 16), 8),
                                  D_IN * 16)], w1_v)
  pltpu.sync_copy(b1_hbm.at[pl.ds(pl.multiple_of(t * 16, 8), 16)], b1_v)

  def _process16(srcv, slotv, nrmv):
    idxg[...] = srcv
    slotg[...] = slotv
    pltpu.sync_copy(x_hbm.at[idxg], rows_v)

    for l in range(16):
      nl = nrmv[l]

      def _b(b, c2, l=l, nl=nl):
        v = rows_v[l, pl.ds(b * 16, 16)]
        rows_v[l, pl.ds(b * 16, 16)] = v * nl
        return c2
      lax.fori_loop(0, 8, _b, 0)
    pltpu.sync_copy(rows_v, agg_sh.at[slotg], add=True)

  ngroups = (nslots + SMAX - 1) // SMAX

  def _group(g, c):
    glo = g * SMAX
    gcount = jnp.minimum(nslots - glo, jnp.int32(SMAX))

    for l in range(16):
      def _zr2(b, c2, l=l):
        rows_v[l, pl.ds(b * 16, 16)] = fzero16
        return c2
      lax.fori_loop(0, 8, _zr2, 0)

    mv = (gcount + 15) // 16

    def _za(k, c2):
      mchunk = k * 16 + t
      rvec = mchunk * 16 + iota
      rz = jnp.where(rvec < gcount, rvec, jnp.int32(SMAX))
      slotg[...] = rz
      pltpu.sync_copy(rows_v, agg_sh.at[slotg])
      return c2
    lax.fori_loop(0, jnp.maximum(0, (mv - t + 15) // 16), _za, 0)
    plsc.subcore_barrier()

    def _chunk_e(cch, cc):
      base = pl.multiple_of((t * NCHUNK + cch) * CHUNK, 8)
      pltpu.sync_copy(dst_hbm.at[pl.ds(base, CHUNK)],
                      dbuf.at[pl.ds(0, CHUNK)])
      pltpu.sync_copy(src_hbm.at[pl.ds(base, CHUNK)], sbuf)

      def _vec(i, mcnt):
        d = dbuf[pl.ds(i * 16, 16)]
        f = plsc.load_gather(flag_v, [d])
        gs = f - 1 - glo
        m = (f > 0) & (gs >= 0) & (gs < gcount)

        def _found(mc):
          s = sbuf[pl.ds(i * 16, 16)]
          nrm = plsc.load_gather(dis_v, [s]) * plsc.load_gather(dis_v, [d])
          pc = plsc.cumsum(m.astype(jnp.int32))
          pos = pc - 1 + mc
          plsc.store_scatter(msrc, [pos], s, mask=m)
          plsc.store_scatter(mslot, [pos], gs, mask=m)
          plsc.store_scatter(mnrm, [pos], nrm, mask=m)
          return mc + pc[15]

        return lax.cond(jnp.any(m), _found, lambda mc: mc, mcnt)

      mcnt = lax.fori_loop(0, VPC, _vec, jnp.int32(0), unroll=2)

      flo = (mcnt // 16) * 16
      padm = (iota + flo) >= mcnt
      plsc.store_scatter(msrc, [iota + flo], izero16, mask=padm)
      plsc.store_scatter(mslot, [iota + flo],
                         jnp.full((16,), SMAX, jnp.int32), mask=padm)
      plsc.store_scatter(mnrm, [iota + flo], fzero16, mask=padm)

      def _bat(r, c2):
        _process16(
            msrc[pl.ds(r * 16, 16)],
            mslot[pl.ds(r * 16, 16)],
            mnrm[pl.ds(r * 16, 16)],
        )
        return c2
      lax.fori_loop(0, (mcnt + 15) // 16, _bat, 0)
      return cc

    lax.fori_loop(0, NCHUNK, _chunk_e, 0)

    gv = (gcount + 15) // 16

    def _selfk(k, c2):
      v = k * 16 + t
      gslot = v * 16 + iota
      jvec = glo + gslot
      m = gslot < gcount
      nodes = plsc.load_gather(slotnode_v, [jvec], mask=m)
      nodes = jnp.where(m, nodes, 0)
      dv = plsc.load_gather(dis_v, [nodes])
      nrm = jnp.where(m, dv * dv, fzero16)
      slots = jnp.where(m, gslot, jnp.int32(SMAX))
      _process16(nodes, slots, nrm)
      return c2
    lax.fori_loop(0, jnp.maximum(0, (gv - t + 15) // 16), _selfk, 0)
    plsc.subcore_barrier()

    def _fb(r0, c2):
      rvec = r0 * 16 + iota
      rz = jnp.where(rvec < gcount, rvec, 0)
      idxg[...] = rz
      pltpu.sync_copy(agg_sh.at[idxg], rows_v)
      zreg = zblk[...]
      for l in range(16):
        acc = b1_v[...]

        def _kv(kv, acc, l=l):
          av = rows_v[l, pl.ds(kv * 16, 16)]
          for lane in range(16):
            acc = acc + av[lane] * w1_v[pl.ds((kv * 16 + lane) * 16, 16)]
          return acc
        acc = lax.fori_loop(0, D_IN // 16, _kv, acc)
        h = jnp.maximum(acc, 0.0)
        j = glo + r0 * 16 + l
        valid = (r0 * 16 + l < gcount).astype(jnp.float32)
        wj = w_v[pl.ds(j, 16)][0]
        wt = (dis0 * wj
              + jnp.where(j == 0, dis0 * dis0, jnp.float32(0.0))) * valid
        zreg = zreg + wt * h
      zblk[...] = zreg
      return c2
    lax.fori_loop(0, (gcount + 15) // 16, _fb, 0)
    plsc.subcore_barrier()
    return c

  lax.fori_loop(0, ngroups, _group, 0)

  pltpu.sync_copy(zblk, z_sh.at[pl.ds(pl.multiple_of(t * 16, 8), 16)])
  plsc.subcore_barrier()

  @pl.when(t == 0)
  def _finish():
    pltpu.sync_copy(z_sh, zfull)
    pltpu.sync_copy(zfull, z_hbm)


_sc_kernel = pl.kernel(
    _sc_body,
    out_type=(
        jax.ShapeDtypeStruct((D_H,), jnp.float32),
        jax.ShapeDtypeStruct((E,), jnp.int32),
        jax.ShapeDtypeStruct((T * NPAD,), jnp.float32),
    ),
    mesh=_mesh,
    compiler_params=pltpu.CompilerParams(needs_layout_passes=False),
    scratch_types=[
        pltpu.VMEM((MCAP,), jnp.int32),
        pltpu.VMEM((CHUNK,), jnp.int32),
        pltpu.VMEM((N + 16,), jnp.float32),
        pltpu.VMEM((N + 16,), jnp.int32),
        pltpu.VMEM((EPT,), jnp.int32),
        pltpu.VMEM((SCAP,), jnp.int32),
        pltpu.VMEM((SCAP,), jnp.float32),
        pltpu.VMEM((MCAP,), jnp.int32),
        pltpu.VMEM((MCAP,), jnp.int32),
        pltpu.VMEM((MCAP,), jnp.float32),
        pltpu.VMEM((16,), jnp.int32),
        pltpu.VMEM((16,), jnp.int32),
        pltpu.VMEM((16, D_IN), jnp.float32),
        pltpu.VMEM((D_IN * 16,), jnp.float32),
        pltpu.VMEM((16,), jnp.float32),
        pltpu.VMEM((16,), jnp.float32),
        pltpu.VMEM((D_H,), jnp.float32),
        pltpu.VMEM((16,), jnp.int32),
        pltpu.VMEM((T * 16,), jnp.int32),
        pltpu.VMEM((640,), jnp.float32),
        pltpu.VMEM((T * 640,), jnp.float32),
        pltpu.SemaphoreType.DMA,
        pltpu.VMEM_SHARED((NPAD,), jnp.float32),
        pltpu.VMEM_SHARED((N,), jnp.int32),
        pltpu.VMEM_SHARED((SCAP,), jnp.int32),
        pltpu.VMEM_SHARED((SCAP,), jnp.float32),
        pltpu.VMEM_SHARED((16,), jnp.int32),
        pltpu.VMEM_SHARED((T * 16,), jnp.int32),
        pltpu.VMEM_SHARED((SMAX + 8, D_IN), jnp.float32),
        pltpu.VMEM_SHARED((D_H,), jnp.float32),
    ],
)


def _head_body(z_ref, w2_ref, b2_ref, wh1_ref, bh1_ref, wh2_ref, bh2_ref,
               o_ref):
  z = z_ref[...]
  h2 = jnp.maximum(
      jnp.dot(z, w2_ref[...], preferred_element_type=jnp.float32)
      + b2_ref[...], 0.0)
  hid = jnp.maximum(
      jnp.dot(h2, wh1_ref[...], preferred_element_type=jnp.float32)
      + bh1_ref[...], 0.0)
  o_ref[...] = (
      jnp.dot(hid, wh2_ref[...], preferred_element_type=jnp.float32)
      + bh2_ref[...])


_head_call = pl.pallas_call(
    _head_body,
    out_shape=jax.ShapeDtypeStruct((1, D_OUT), jnp.float32),
)


def kernel(x, edge_index, W1, b1, W2, b2, Wh1, bh1, Wh2, bh2):
  w1_blocks = W1.reshape(D_IN, 16, 16).transpose(1, 0, 2).reshape(-1)
  z, _, _ = _sc_kernel(edge_index[0], edge_index[1], x, w1_blocks, b1)
  q = _head_call(
      z.reshape(1, D_H), W2, b2.reshape(1, D_H),
      Wh1, bh1.reshape(1, D_H), Wh2, bh2.reshape(1, D_OUT))
  return q.reshape(D_OUT)


# speedup vs baseline: 72.9233x; 1.1626x over previous
import jax
import jax.numpy as jnp
from jax import lax
from jax.experimental import pallas as pl
from jax.experimental.pallas import tpu as pltpu
from jax.experimental.pallas import tpu_sc as plsc

N = 10000
E = 320000
D_IN = 128
D_H = 256
D_OUT = 64

T = 16
EPT = E // T
CHUNK = 2000
NCHUNK = EPT // CHUNK
VPC = CHUNK // 16
SCAP = N + 16
NVEC = N // 16
MCAP = CHUNK + 16
NPAD = 10240
SMAX = 1024

_mesh = plsc.VectorSubcoreMesh(
    core_axis_name="c", subcore_axis_name="s", num_cores=1, num_subcores=T
)


def _rsqrt(x):
  i = plsc.bitcast(x, jnp.int32)
  y = plsc.bitcast(jnp.int32(0x5F3759DF) - (i >> 1), jnp.float32)
  for _ in range(4):
    y = y * (1.5 - 0.5 * x * y * y)
  return y


def _sc_body(
    ei_hbm, x_hbm, w1_hbm, b1_hbm,
    z_hbm, l0_hbm, hist_hbm,
    dbuf, sbuf, dbuf2, sbuf2, sem0, sem1, dis_v, flag_v, l0buf, slotnode_v, w_v,
    msrc, mslot, mnrm, idxg, slotg, rows_v,
    w1_v, b1_v, zblk, zfull, vec16, cntall_v, degbuf, hbuf, hsem,
    dis_sh, flag_sh, slotnode_sh, w_sh, meta_sh, cnt_sh, agg_sh, z_sh,
):
  t = lax.axis_index("s")
  iota = lax.iota(jnp.int32, 16)
  fzero16 = jnp.zeros((16,), jnp.float32)
  izero16 = jnp.zeros((16,), jnp.int32)

  def _z(i, c):
    dis_v[pl.ds(i * 16, 16)] = fzero16
    flag_v[pl.ds(i * 16, 16)] = izero16
    return c
  lax.fori_loop(0, NVEC, _z, 0)

  def _z2(i, c):
    w_v[pl.ds(i * 16, 16)] = fzero16
    slotnode_v[pl.ds(i * 16, 16)] = izero16
    return c
  lax.fori_loop(0, SCAP // 16, _z2, 0)

  for l in range(16):
    def _zr(b, c, l=l):
      rows_v[l, pl.ds(b * 16, 16)] = fzero16
      return c
    lax.fori_loop(0, 8, _zr, 0)
  zblk[...] = fzero16

  def _edma(cidx, db, sb, sem):
    base = pl.multiple_of((t * NCHUNK + cidx) * CHUNK, 8)
    d1 = pltpu.make_async_copy(
        ei_hbm.at[pl.ds(E + base, CHUNK)], db.at[pl.ds(0, CHUNK)], sem)
    d2 = pltpu.make_async_copy(ei_hbm.at[pl.ds(base, CHUNK)], sb, sem)
    return d1, d2

  def _fire(cidx, db, sb, sem):
    d1, d2 = _edma(cidx, db, sb, sem)
    d1.start()
    d2.start()

  def _drain(cidx, db, sb, sem):
    d1, d2 = _edma(cidx, db, sb, sem)
    d1.wait()
    d2.wait()

  def _scan_pipe(chunk_body, init):
    _fire(0, dbuf, sbuf, sem0)
    _fire(1, dbuf2, sbuf2, sem1)

    def _pair(pp, carry):
      c0 = pp * 2
      _drain(c0, dbuf, sbuf, sem0)
      carry = chunk_body(dbuf, sbuf, carry)
      _fire(c0 + 2, dbuf, sbuf, sem0)
      _drain(c0 + 1, dbuf2, sbuf2, sem1)
      carry = chunk_body(dbuf2, sbuf2, carry)
      _fire(c0 + 3, dbuf2, sbuf2, sem1)
      return carry
    carry = lax.fori_loop(0, NCHUNK // 2 - 1, _pair, init)
    _drain(NCHUNK - 2, dbuf, sbuf, sem0)
    carry = chunk_body(dbuf, sbuf, carry)
    _drain(NCHUNK - 1, dbuf2, sbuf2, sem1)
    carry = chunk_body(dbuf2, sbuf2, carry)
    return carry

  def _chunk_a(db, sb, cnt0):
    def _vec(i, cnt0):
      d = db[pl.ds(i * 16, 16)]
      cntv, lastm = plsc.scan_count(d)
      plsc.addupdate_scatter(
          dis_v, [d], cntv.astype(jnp.float32), mask=lastm)
      m = d == 0

      def _found(cc):
        s = sb[pl.ds(i * 16, 16)]
        pc = plsc.cumsum(m.astype(jnp.int32))
        pos = pc - 1 + cc
        plsc.store_scatter(l0buf, [pos], s, mask=m)
        return cc + pc[15]

      return lax.cond(jnp.any(m), _found, lambda cc: cc, cnt0)

    return lax.fori_loop(0, VPC, _vec, cnt0, unroll=2)

  cnt0 = _scan_pipe(_chunk_a, jnp.int32(0))

  pltpu.sync_copy(dis_v.at[pl.ds(0, N)],
                  hist_hbm.at[pl.ds(pl.multiple_of(t * NPAD, 8), N)])
  pltpu.sync_copy(l0buf, l0_hbm.at[pl.ds(pl.multiple_of(t * EPT, 8), EPT)])
  vec16[...] = jnp.full((16,), cnt0, jnp.int32)
  pltpu.sync_copy(vec16, cnt_sh.at[pl.ds(pl.multiple_of(t * 16, 8), 16)])
  plsc.subcore_barrier()

  copies = [
      pltpu.make_async_copy(
          hist_hbm.at[pl.ds(pl.multiple_of(tt * NPAD + t * 640, 8), 640)],
          hbuf.at[pl.ds(tt * 640, 640)], hsem)
      for tt in range(T)
  ]
  for cp in copies:
    cp.start()
  for cp in copies:
    cp.wait()

  def _acc(i, c2):
    acc = hbuf[pl.ds(i * 16, 16)]
    for tt in range(1, T):
      acc = acc + hbuf[pl.ds(tt * 640 + i * 16, 16)]
    degbuf[pl.ds(i * 16, 16)] = acc
    return c2
  lax.fori_loop(0, 40, _acc, 0)

  def _dis(i, c):
    dv = degbuf[pl.ds(i * 16, 16)] + 1.0
    degbuf[pl.ds(i * 16, 16)] = _rsqrt(dv)
    return c
  lax.fori_loop(0, 40, _dis, 0)
  pltpu.sync_copy(degbuf, dis_sh.at[pl.ds(pl.multiple_of(t * 640, 8), 640)])
  plsc.subcore_barrier()
  pltpu.sync_copy(dis_sh.at[pl.ds(0, N)], dis_v.at[pl.ds(0, N)])

  lane0 = iota == 0

  def _sstore(ref, idx, val):
    plsc.store_scatter(
        ref, [jnp.full((16,), idx, jnp.int32)],
        jnp.full((16,), val, ref.dtype), mask=lane0)

  @pl.when(t == 0)
  def _dedup():
    pltpu.sync_copy(cnt_sh, cntall_v)
    _sstore(flag_v, jnp.int32(0), jnp.int32(1))

    def _tile(tt, ns):
      cnt_t = cntall_v[pl.ds(tt * 16, 16)][0]

      def _chunk(c, ns):
        cbase = pl.multiple_of((tt * NCHUNK + c) * CHUNK, 8)
        pltpu.sync_copy(l0_hbm.at[pl.ds(cbase, CHUNK)],
                        dbuf.at[pl.ds(0, CHUNK)])
        kmax = jnp.minimum(jnp.int32(CHUNK), cnt_t - c * CHUNK)

        def _k(k, ns):
          s = dbuf[pl.ds(k, 16)][0]
          f = flag_v[pl.ds(s, 16)][0]
          isnew = (f == 0).astype(jnp.int32)
          slot = jnp.where(f == 0, ns, f - 1)
          _sstore(flag_v, s, slot + 1)
          _sstore(slotnode_v, slot, s)
          wnew = w_v[pl.ds(slot, 16)][0] + dis_v[pl.ds(s, 16)][0]
          _sstore(w_v, slot, wnew)
          return ns + isnew

        return lax.fori_loop(0, kmax, _k, ns)

      nchunks = (cnt_t + CHUNK - 1) // CHUNK
      return lax.fori_loop(0, nchunks, _chunk, ns)

    ns = lax.fori_loop(0, T, _tile, jnp.int32(1))
    pltpu.sync_copy(flag_v.at[pl.ds(0, N)], flag_sh)
    pltpu.sync_copy(slotnode_v, slotnode_sh)
    pltpu.sync_copy(w_v, w_sh)
    vec16[...] = jnp.full((16,), ns, jnp.int32)
    pltpu.sync_copy(vec16, meta_sh)

  plsc.subcore_barrier()

  pltpu.sync_copy(flag_sh, flag_v.at[pl.ds(0, N)])
  pltpu.sync_copy(slotnode_sh, slotnode_v)
  pltpu.sync_copy(w_sh, w_v)
  pltpu.sync_copy(meta_sh, vec16)
  nslots = vec16[...][0]
  dis0 = dis_v[pl.ds(0, 16)][0]
  pltpu.sync_copy(w1_hbm.at[pl.ds(pl.multiple_of(t * (D_IN * 16), 8),
                                  D_IN * 16)], w1_v)
  pltpu.sync_copy(b1_hbm.at[pl.ds(pl.multiple_of(t * 16, 8), 16)], b1_v)

  def _process16(srcv, slotv, nrmv):
    idxg[...] = srcv
    slotg[...] = slotv
    pltpu.sync_copy(x_hbm.at[idxg], rows_v)

    for l in range(16):
      nl = nrmv[l]

      def _b(b, c2, l=l, nl=nl):
        v = rows_v[l, pl.ds(b * 16, 16)]
        rows_v[l, pl.ds(b * 16, 16)] = v * nl
        return c2
      lax.fori_loop(0, 8, _b, 0)
    pltpu.sync_copy(rows_v, agg_sh.at[slotg], add=True)

  ngroups = (nslots + SMAX - 1) // SMAX

  def _group(g, c):
    glo = g * SMAX
    gcount = jnp.minimum(nslots - glo, jnp.int32(SMAX))

    for l in range(16):
      def _zr2(b, c2, l=l):
        rows_v[l, pl.ds(b * 16, 16)] = fzero16
        return c2
      lax.fori_loop(0, 8, _zr2, 0)

    mv = (gcount + 15) // 16

    def _za(k, c2):
      mchunk = k * 16 + t
      rvec = mchunk * 16 + iota
      rz = jnp.where(rvec < gcount, rvec, jnp.int32(SMAX))
      slotg[...] = rz
      pltpu.sync_copy(rows_v, agg_sh.at[slotg])
      return c2
    lax.fori_loop(0, jnp.maximum(0, (mv - t + 15) // 16), _za, 0)
    plsc.subcore_barrier()

    def _chunk_e(db, sb, cc):
      def _vec(i, mcnt):
        d = db[pl.ds(i * 16, 16)]
        f = plsc.load_gather(flag_v, [d])
        gs = f - 1 - glo
        m = (f > 0) & (gs >= 0) & (gs < gcount)

        def _found(mc):
          s = sb[pl.ds(i * 16, 16)]
          nrm = plsc.load_gather(dis_v, [s]) * plsc.load_gather(dis_v, [d])
          pc = plsc.cumsum(m.astype(jnp.int32))
          pos = pc - 1 + mc
          plsc.store_scatter(msrc, [pos], s, mask=m)
          plsc.store_scatter(mslot, [pos], gs, mask=m)
          plsc.store_scatter(mnrm, [pos], nrm, mask=m)
          return mc + pc[15]

        return lax.cond(jnp.any(m), _found, lambda mc: mc, mcnt)

      mcnt = lax.fori_loop(0, VPC, _vec, jnp.int32(0), unroll=2)

      flo = (mcnt // 16) * 16
      padm = (iota + flo) >= mcnt
      plsc.store_scatter(msrc, [iota + flo], izero16, mask=padm)
      plsc.store_scatter(mslot, [iota + flo],
                         jnp.full((16,), SMAX, jnp.int32), mask=padm)
      plsc.store_scatter(mnrm, [iota + flo], fzero16, mask=padm)

      def _bat(r, c2):
        _process16(
            msrc[pl.ds(r * 16, 16)],
            mslot[pl.ds(r * 16, 16)],
            mnrm[pl.ds(r * 16, 16)],
        )
        return c2
      lax.fori_loop(0, (mcnt + 15) // 16, _bat, 0)
      return cc

    _scan_pipe(_chunk_e, 0)

    gv = (gcount + 15) // 16

    def _selfk(k, c2):
      v = k * 16 + t
      gslot = v * 16 + iota
      jvec = glo + gslot
      m = gslot < gcount
      nodes = plsc.load_gather(slotnode_v, [jvec], mask=m)
      nodes = jnp.where(m, nodes, 0)
      dv = plsc.load_gather(dis_v, [nodes])
      nrm = jnp.where(m, dv * dv, fzero16)
      slots = jnp.where(m, gslot, jnp.int32(SMAX))
      _process16(nodes, slots, nrm)
      return c2
    lax.fori_loop(0, jnp.maximum(0, (gv - t + 15) // 16), _selfk, 0)
    plsc.subcore_barrier()

    def _fb(r0, c2):
      rvec = r0 * 16 + iota
      rz = jnp.where(rvec < gcount, rvec, 0)
      idxg[...] = rz
      pltpu.sync_copy(agg_sh.at[idxg], rows_v)
      zreg = zblk[...]
      for l in range(16):
        acc = b1_v[...]

        def _kv(kv, acc, l=l):
          av = rows_v[l, pl.ds(kv * 16, 16)]
          for lane in range(16):
            acc = acc + av[lane] * w1_v[pl.ds((kv * 16 + lane) * 16, 16)]
          return acc
        acc = lax.fori_loop(0, D_IN // 16, _kv, acc)
        h = jnp.maximum(acc, 0.0)
        j = glo + r0 * 16 + l
        valid = (r0 * 16 + l < gcount).astype(jnp.float32)
        wj = w_v[pl.ds(j, 16)][0]
        wt = (dis0 * wj
              + jnp.where(j == 0, dis0 * dis0, jnp.float32(0.0))) * valid
        zreg = zreg + wt * h
      zblk[...] = zreg
      return c2
    lax.fori_loop(0, (gcount + 15) // 16, _fb, 0)
    plsc.subcore_barrier()
    return c

  lax.fori_loop(0, ngroups, _group, 0)

  pltpu.sync_copy(zblk, z_sh.at[pl.ds(pl.multiple_of(t * 16, 8), 16)])
  plsc.subcore_barrier()

  @pl.when(t == 0)
  def _finish():
    pltpu.sync_copy(z_sh, zfull)
    pltpu.sync_copy(zfull, z_hbm)


_sc_kernel = pl.kernel(
    _sc_body,
    out_type=(
        jax.ShapeDtypeStruct((D_H,), jnp.float32),
        jax.ShapeDtypeStruct((E,), jnp.int32),
        jax.ShapeDtypeStruct((T * NPAD,), jnp.float32),
    ),
    mesh=_mesh,
    compiler_params=pltpu.CompilerParams(needs_layout_passes=False),
    scratch_types=[
        pltpu.VMEM((MCAP,), jnp.int32),
        pltpu.VMEM((CHUNK,), jnp.int32),
        pltpu.VMEM((MCAP,), jnp.int32),
        pltpu.VMEM((CHUNK,), jnp.int32),
        pltpu.SemaphoreType.DMA,
        pltpu.SemaphoreType.DMA,
        pltpu.VMEM((N + 16,), jnp.float32),
        pltpu.VMEM((N + 16,), jnp.int32),
        pltpu.VMEM((EPT,), jnp.int32),
        pltpu.VMEM((SCAP,), jnp.int32),
        pltpu.VMEM((SCAP,), jnp.float32),
        pltpu.VMEM((MCAP,), jnp.int32),
        pltpu.VMEM((MCAP,), jnp.int32),
        pltpu.VMEM((MCAP,), jnp.float32),
        pltpu.VMEM((16,), jnp.int32),
        pltpu.VMEM((16,), jnp.int32),
        pltpu.VMEM((16, D_IN), jnp.float32),
        pltpu.VMEM((D_IN * 16,), jnp.float32),
        pltpu.VMEM((16,), jnp.float32),
        pltpu.VMEM((16,), jnp.float32),
        pltpu.VMEM((D_H,), jnp.float32),
        pltpu.VMEM((16,), jnp.int32),
        pltpu.VMEM((T * 16,), jnp.int32),
        pltpu.VMEM((640,), jnp.float32),
        pltpu.VMEM((T * 640,), jnp.float32),
        pltpu.SemaphoreType.DMA,
        pltpu.VMEM_SHARED((NPAD,), jnp.float32),
        pltpu.VMEM_SHARED((N,), jnp.int32),
        pltpu.VMEM_SHARED((SCAP,), jnp.int32),
        pltpu.VMEM_SHARED((SCAP,), jnp.float32),
        pltpu.VMEM_SHARED((16,), jnp.int32),
        pltpu.VMEM_SHARED((T * 16,), jnp.int32),
        pltpu.VMEM_SHARED((SMAX + 8, D_IN), jnp.float32),
        pltpu.VMEM_SHARED((D_H,), jnp.float32),
    ],
)


def _head_body(z_ref, w2_ref, b2_ref, wh1_ref, bh1_ref, wh2_ref, bh2_ref,
               o_ref):
  z = z_ref[...]
  h2 = jnp.maximum(
      jnp.dot(z, w2_ref[...], preferred_element_type=jnp.float32)
      + b2_ref[...], 0.0)
  hid = jnp.maximum(
      jnp.dot(h2, wh1_ref[...], preferred_element_type=jnp.float32)
      + bh1_ref[...], 0.0)
  o_ref[...] = (
      jnp.dot(hid, wh2_ref[...], preferred_element_type=jnp.float32)
      + bh2_ref[...])


_head_call = pl.pallas_call(
    _head_body,
    out_shape=jax.ShapeDtypeStruct((1, D_OUT), jnp.float32),
)


def kernel(x, edge_index, W1, b1, W2, b2, Wh1, bh1, Wh2, bh2):
  w1_blocks = W1.reshape(D_IN, 16, 16).transpose(1, 0, 2).reshape(-1)
  z, _, _ = _sc_kernel(edge_index.reshape(-1), x, w1_blocks, b1)
  q = _head_call(
      z.reshape(1, D_H), W2, b2.reshape(1, D_H),
      Wh1, bh1.reshape(1, D_H), Wh2, bh2.reshape(1, D_OUT))
  return q.reshape(D_OUT)


# speedup vs baseline: 80.2248x; 1.1001x over previous
import jax
import jax.numpy as jnp
from jax import lax
from jax.experimental import pallas as pl
from jax.experimental.pallas import tpu as pltpu
from jax.experimental.pallas import tpu_sc as plsc

N = 10000
E = 320000
D_IN = 128
D_H = 256
D_OUT = 64

T = 16
EPT = E // T
CHUNK = 2000
NCHUNK = EPT // CHUNK
VPC = CHUNK // 16
SCAP = N + 16
NVEC = N // 16
MCAP = CHUNK + 16
NPAD = 10240
SMAX = 1024

_mesh = plsc.VectorSubcoreMesh(
    core_axis_name="c", subcore_axis_name="s", num_cores=1, num_subcores=T
)


def _rsqrt(x):
  i = plsc.bitcast(x, jnp.int32)
  y = plsc.bitcast(jnp.int32(0x5F3759DF) - (i >> 1), jnp.float32)
  for _ in range(4):
    y = y * (1.5 - 0.5 * x * y * y)
  return y


def _sc_body(
    ei_hbm, x_hbm, w1_hbm, b1_hbm,
    z_hbm, l0_hbm, hist_hbm,
    dbuf, sbuf, dbuf2, sbuf2, sem0, sem1, dis_v, flag_v, l0buf, slotnode_v, w_v,
    msrc, mslot, mnrm, idxg, slotg, rows_v,
    w1_v, b1_v, zblk, zfull, vec16, cntall_v, degbuf, hbuf, hsem,
    dis_sh, flag_sh, slotnode_sh, w_sh, meta_sh, cnt_sh, agg_sh, z_sh,
):
  t = lax.axis_index("s")
  iota = lax.iota(jnp.int32, 16)
  fzero16 = jnp.zeros((16,), jnp.float32)
  izero16 = jnp.zeros((16,), jnp.int32)

  def _z(i, c):
    dis_v[pl.ds(i * 16, 16)] = fzero16
    flag_v[pl.ds(i * 16, 16)] = izero16
    return c
  lax.fori_loop(0, NVEC, _z, 0)

  def _z2(i, c):
    w_v[pl.ds(i * 16, 16)] = fzero16
    slotnode_v[pl.ds(i * 16, 16)] = izero16
    return c
  lax.fori_loop(0, SCAP // 16, _z2, 0)

  for l in range(16):
    def _zr(b, c, l=l):
      rows_v[l, pl.ds(b * 16, 16)] = fzero16
      return c
    lax.fori_loop(0, 8, _zr, 0)
  zblk[...] = fzero16

  def _edma(cidx, db, sb, sem):
    base = pl.multiple_of((t * NCHUNK + cidx) * CHUNK, 8)
    d1 = pltpu.make_async_copy(
        ei_hbm.at[pl.ds(E + base, CHUNK)], db.at[pl.ds(0, CHUNK)], sem)
    d2 = pltpu.make_async_copy(ei_hbm.at[pl.ds(base, CHUNK)], sb, sem)
    return d1, d2

  def _fire(cidx, db, sb, sem):
    d1, d2 = _edma(cidx, db, sb, sem)
    d1.start()
    d2.start()

  def _drain(cidx, db, sb, sem):
    d1, d2 = _edma(cidx, db, sb, sem)
    d1.wait()
    d2.wait()

  def _scan_pipe(chunk_body, init):
    _fire(0, dbuf, sbuf, sem0)
    _fire(1, dbuf2, sbuf2, sem1)

    def _pair(pp, carry):
      c0 = pp * 2
      _drain(c0, dbuf, sbuf, sem0)
      carry = chunk_body(dbuf, sbuf, carry)
      _fire(c0 + 2, dbuf, sbuf, sem0)
      _drain(c0 + 1, dbuf2, sbuf2, sem1)
      carry = chunk_body(dbuf2, sbuf2, carry)
      _fire(c0 + 3, dbuf2, sbuf2, sem1)
      return carry
    carry = lax.fori_loop(0, NCHUNK // 2 - 1, _pair, init)
    _drain(NCHUNK - 2, dbuf, sbuf, sem0)
    carry = chunk_body(dbuf, sbuf, carry)
    _drain(NCHUNK - 1, dbuf2, sbuf2, sem1)
    carry = chunk_body(dbuf2, sbuf2, carry)
    return carry


  def _chunk_a(db, sb, cnt0v):
    def _vec(i, cv):
      d = db[pl.ds(i * 16, 16)]
      cntv, lastm = plsc.scan_count(d)
      plsc.addupdate_scatter(
          dis_v, [d], cntv.astype(jnp.float32), mask=lastm)
      m = d == 0
      s = sb[pl.ds(i * 16, 16)]
      pc = plsc.cumsum(m.astype(jnp.int32))
      pos = pc - 1 + cv
      plsc.store_scatter(l0buf, [pos], s, mask=m)
      return cv + plsc.all_reduce_population_count(m)

    return lax.fori_loop(0, VPC, _vec, cnt0v, unroll=2)

  cnt0v = _scan_pipe(_chunk_a, izero16)
  cnt0 = cnt0v[0]

  pltpu.sync_copy(dis_v.at[pl.ds(0, N)],
                  hist_hbm.at[pl.ds(pl.multiple_of(t * NPAD, 8), N)])
  pltpu.sync_copy(l0buf, l0_hbm.at[pl.ds(pl.multiple_of(t * EPT, 8), EPT)])
  vec16[...] = jnp.full((16,), cnt0, jnp.int32)
  pltpu.sync_copy(vec16, cnt_sh.at[pl.ds(pl.multiple_of(t * 16, 8), 16)])
  plsc.subcore_barrier()

  copies = [
      pltpu.make_async_copy(
          hist_hbm.at[pl.ds(pl.multiple_of(tt * NPAD + t * 640, 8), 640)],
          hbuf.at[pl.ds(tt * 640, 640)], hsem)
      for tt in range(T)
  ]
  for cp in copies:
    cp.start()
  for cp in copies:
    cp.wait()

  def _acc(i, c2):
    acc = hbuf[pl.ds(i * 16, 16)]
    for tt in range(1, T):
      acc = acc + hbuf[pl.ds(tt * 640 + i * 16, 16)]
    degbuf[pl.ds(i * 16, 16)] = acc
    return c2
  lax.fori_loop(0, 40, _acc, 0)

  def _dis(i, c):
    dv = degbuf[pl.ds(i * 16, 16)] + 1.0
    degbuf[pl.ds(i * 16, 16)] = _rsqrt(dv)
    return c
  lax.fori_loop(0, 40, _dis, 0)
  pltpu.sync_copy(degbuf, dis_sh.at[pl.ds(pl.multiple_of(t * 640, 8), 640)])
  plsc.subcore_barrier()
  pltpu.sync_copy(dis_sh.at[pl.ds(0, N)], dis_v.at[pl.ds(0, N)])

  lane0 = iota == 0

  def _sstore(ref, idx, val):
    plsc.store_scatter(
        ref, [jnp.full((16,), idx, jnp.int32)],
        jnp.full((16,), val, ref.dtype), mask=lane0)

  @pl.when(t == 0)
  def _dedup():
    pltpu.sync_copy(cnt_sh, cntall_v)
    _sstore(flag_v, jnp.int32(0), jnp.int32(1))

    def _tile(tt, ns):
      cnt_t = cntall_v[pl.ds(tt * 16, 16)][0]

      def _chunk(c, ns):
        cbase = pl.multiple_of((tt * NCHUNK + c) * CHUNK, 8)
        pltpu.sync_copy(l0_hbm.at[pl.ds(cbase, CHUNK)],
                        dbuf.at[pl.ds(0, CHUNK)])
        kmax = jnp.minimum(jnp.int32(CHUNK), cnt_t - c * CHUNK)

        def _k(k, ns):
          s = dbuf[pl.ds(k, 16)][0]
          f = flag_v[pl.ds(s, 16)][0]
          isnew = (f == 0).astype(jnp.int32)
          slot = jnp.where(f == 0, ns, f - 1)
          _sstore(flag_v, s, slot + 1)
          _sstore(slotnode_v, slot, s)
          wnew = w_v[pl.ds(slot, 16)][0] + dis_v[pl.ds(s, 16)][0]
          _sstore(w_v, slot, wnew)
          return ns + isnew

        return lax.fori_loop(0, kmax, _k, ns)

      nchunks = (cnt_t + CHUNK - 1) // CHUNK
      return lax.fori_loop(0, nchunks, _chunk, ns)

    ns = lax.fori_loop(0, T, _tile, jnp.int32(1))
    pltpu.sync_copy(flag_v.at[pl.ds(0, N)], flag_sh)
    pltpu.sync_copy(slotnode_v, slotnode_sh)
    pltpu.sync_copy(w_v, w_sh)
    vec16[...] = jnp.full((16,), ns, jnp.int32)
    pltpu.sync_copy(vec16, meta_sh)

  plsc.subcore_barrier()

  pltpu.sync_copy(flag_sh, flag_v.at[pl.ds(0, N)])
  pltpu.sync_copy(slotnode_sh, slotnode_v)
  pltpu.sync_copy(w_sh, w_v)
  pltpu.sync_copy(meta_sh, vec16)
  nslots = vec16[...][0]
  dis0 = dis_v[pl.ds(0, 16)][0]
  pltpu.sync_copy(w1_hbm.at[pl.ds(pl.multiple_of(t * (D_IN * 16), 8),
                                  D_IN * 16)], w1_v)
  pltpu.sync_copy(b1_hbm.at[pl.ds(pl.multiple_of(t * 16, 8), 16)], b1_v)

  def _process16(srcv, slotv, nrmv):
    idxg[...] = srcv
    slotg[...] = slotv
    pltpu.sync_copy(x_hbm.at[idxg], rows_v)

    for l in range(16):
      nl = nrmv[l]

      def _b(b, c2, l=l, nl=nl):
        v = rows_v[l, pl.ds(b * 16, 16)]
        rows_v[l, pl.ds(b * 16, 16)] = v * nl
        return c2
      lax.fori_loop(0, 8, _b, 0)
    pltpu.sync_copy(rows_v, agg_sh.at[slotg], add=True)

  ngroups = (nslots + SMAX - 1) // SMAX

  def _group(g, c):
    glo = g * SMAX
    gcount = jnp.minimum(nslots - glo, jnp.int32(SMAX))

    for l in range(16):
      def _zr2(b, c2, l=l):
        rows_v[l, pl.ds(b * 16, 16)] = fzero16
        return c2
      lax.fori_loop(0, 8, _zr2, 0)

    mv = (gcount + 15) // 16

    def _za(k, c2):
      mchunk = k * 16 + t
      rvec = mchunk * 16 + iota
      rz = jnp.where(rvec < gcount, rvec, jnp.int32(SMAX))
      slotg[...] = rz
      pltpu.sync_copy(rows_v, agg_sh.at[slotg])
      return c2
    lax.fori_loop(0, jnp.maximum(0, (mv - t + 15) // 16), _za, 0)
    plsc.subcore_barrier()

    def _chunk_e(db, sb, cc):
      def _vec(i, mcv):
        d = db[pl.ds(i * 16, 16)]
        f = plsc.load_gather(flag_v, [d])
        gs = f - 1 - glo
        m = (f > 0) & (gs >= 0) & (gs < gcount)
        s = sb[pl.ds(i * 16, 16)]
        nrm = plsc.load_gather(dis_v, [s]) * plsc.load_gather(dis_v, [d])
        pc = plsc.cumsum(m.astype(jnp.int32))
        pos = pc - 1 + mcv
        plsc.store_scatter(msrc, [pos], s, mask=m)
        plsc.store_scatter(mslot, [pos], gs, mask=m)
        plsc.store_scatter(mnrm, [pos], nrm, mask=m)
        return mcv + plsc.all_reduce_population_count(m)

      mcv = lax.fori_loop(0, VPC, _vec, izero16, unroll=2)
      mcnt = mcv[0]

      flo = (mcnt // 16) * 16
      padm = (iota + flo) >= mcnt
      plsc.store_scatter(msrc, [iota + flo], izero16, mask=padm)
      plsc.store_scatter(mslot, [iota + flo],
                         jnp.full((16,), SMAX, jnp.int32), mask=padm)
      plsc.store_scatter(mnrm, [iota + flo], fzero16, mask=padm)

      def _bat(r, c2):
        _process16(
            msrc[pl.ds(r * 16, 16)],
            mslot[pl.ds(r * 16, 16)],
            mnrm[pl.ds(r * 16, 16)],
        )
        return c2
      lax.fori_loop(0, (mcnt + 15) // 16, _bat, 0)
      return cc

    _scan_pipe(_chunk_e, 0)

    gv = (gcount + 15) // 16

    def _selfk(k, c2):
      v = k * 16 + t
      gslot = v * 16 + iota
      jvec = glo + gslot
      m = gslot < gcount
      nodes = plsc.load_gather(slotnode_v, [jvec], mask=m)
      nodes = jnp.where(m, nodes, 0)
      dv = plsc.load_gather(dis_v, [nodes])
      nrm = jnp.where(m, dv * dv, fzero16)
      slots = jnp.where(m, gslot, jnp.int32(SMAX))
      _process16(nodes, slots, nrm)
      return c2
    lax.fori_loop(0, jnp.maximum(0, (gv - t + 15) // 16), _selfk, 0)
    plsc.subcore_barrier()

    def _fb(r0, c2):
      rvec = r0 * 16 + iota
      rz = jnp.where(rvec < gcount, rvec, 0)
      idxg[...] = rz
      pltpu.sync_copy(agg_sh.at[idxg], rows_v)
      zreg = zblk[...]
      for l in range(16):
        acc = b1_v[...]

        def _kv(kv, acc, l=l):
          av = rows_v[l, pl.ds(kv * 16, 16)]
          for lane in range(16):
            acc = acc + av[lane] * w1_v[pl.ds((kv * 16 + lane) * 16, 16)]
          return acc
        acc = lax.fori_loop(0, D_IN // 16, _kv, acc)
        h = jnp.maximum(acc, 0.0)
        j = glo + r0 * 16 + l
        valid = (r0 * 16 + l < gcount).astype(jnp.float32)
        wj = w_v[pl.ds(j, 16)][0]
        wt = (dis0 * wj
              + jnp.where(j == 0, dis0 * dis0, jnp.float32(0.0))) * valid
        zreg = zreg + wt * h
      zblk[...] = zreg
      return c2
    lax.fori_loop(0, (gcount + 15) // 16, _fb, 0)
    plsc.subcore_barrier()
    return c

  lax.fori_loop(0, ngroups, _group, 0)

  pltpu.sync_copy(zblk, z_sh.at[pl.ds(pl.multiple_of(t * 16, 8), 16)])
  plsc.subcore_barrier()

  @pl.when(t == 0)
  def _finish():
    pltpu.sync_copy(z_sh, zfull)
    pltpu.sync_copy(zfull, z_hbm)


_sc_kernel = pl.kernel(
    _sc_body,
    out_type=(
        jax.ShapeDtypeStruct((D_H,), jnp.float32),
        jax.ShapeDtypeStruct((E,), jnp.int32),
        jax.ShapeDtypeStruct((T * NPAD,), jnp.float32),
    ),
    mesh=_mesh,
    compiler_params=pltpu.CompilerParams(needs_layout_passes=False),
    scratch_types=[
        pltpu.VMEM((MCAP,), jnp.int32),
        pltpu.VMEM((CHUNK,), jnp.int32),
        pltpu.VMEM((MCAP,), jnp.int32),
        pltpu.VMEM((CHUNK,), jnp.int32),
        pltpu.SemaphoreType.DMA,
        pltpu.SemaphoreType.DMA,
        pltpu.VMEM((N + 16,), jnp.float32),
        pltpu.VMEM((N + 16,), jnp.int32),
        pltpu.VMEM((EPT,), jnp.int32),
        pltpu.VMEM((SCAP,), jnp.int32),
        pltpu.VMEM((SCAP,), jnp.float32),
        pltpu.VMEM((MCAP,), jnp.int32),
        pltpu.VMEM((MCAP,), jnp.int32),
        pltpu.VMEM((MCAP,), jnp.float32),
        pltpu.VMEM((16,), jnp.int32),
        pltpu.VMEM((16,), jnp.int32),
        pltpu.VMEM((16, D_IN), jnp.float32),
        pltpu.VMEM((D_IN * 16,), jnp.float32),
        pltpu.VMEM((16,), jnp.float32),
        pltpu.VMEM((16,), jnp.float32),
        pltpu.VMEM((D_H,), jnp.float32),
        pltpu.VMEM((16,), jnp.int32),
        pltpu.VMEM((T * 16,), jnp.int32),
        pltpu.VMEM((640,), jnp.float32),
        pltpu.VMEM((T * 640,), jnp.float32),
        pltpu.SemaphoreType.DMA,
        pltpu.VMEM_SHARED((NPAD,), jnp.float32),
        pltpu.VMEM_SHARED((N,), jnp.int32),
        pltpu.VMEM_SHARED((SCAP,), jnp.int32),
        pltpu.VMEM_SHARED((SCAP,), jnp.float32),
        pltpu.VMEM_SHARED((16,), jnp.int32),
        pltpu.VMEM_SHARED((T * 16,), jnp.int32),
        pltpu.VMEM_SHARED((SMAX + 8, D_IN), jnp.float32),
        pltpu.VMEM_SHARED((D_H,), jnp.float32),
    ],
)


def _head_body(z_ref, w2_ref, b2_ref, wh1_ref, bh1_ref, wh2_ref, bh2_ref,
               o_ref):
  z = z_ref[...]
  h2 = jnp.maximum(
      jnp.dot(z, w2_ref[...], preferred_element_type=jnp.float32)
      + b2_ref[...], 0.0)
  hid = jnp.maximum(
      jnp.dot(h2, wh1_ref[...], preferred_element_type=jnp.float32)
      + bh1_ref[...], 0.0)
  o_ref[...] = (
      jnp.dot(hid, wh2_ref[...], preferred_element_type=jnp.float32)
      + bh2_ref[...])


_head_call = pl.pallas_call(
    _head_body,
    out_shape=jax.ShapeDtypeStruct((1, D_OUT), jnp.float32),
)


def kernel(x, edge_index, W1, b1, W2, b2, Wh1, bh1, Wh2, bh2):
  w1_blocks = W1.reshape(D_IN, 16, 16).transpose(1, 0, 2).reshape(-1)
  z, _, _ = _sc_kernel(edge_index.reshape(-1), x, w1_blocks, b1)
  q = _head_call(
      z.reshape(1, D_H), W2, b2.reshape(1, D_H),
      Wh1, bh1.reshape(1, D_H), Wh2, bh2.reshape(1, D_OUT))
  return q.reshape(D_OUT)
